# Initial kernel scaffold; baseline (speedup 1.0000x reference)
import jax
import jax.numpy as jnp
from jax.experimental import pallas as pl


def kernel(x, edge_index, W1, b1, bn1_g, bn1_b, W2, b2, bn2_g, bn2_b, Wres, bres, ln_g, ln_b, W3, b3):
    raise NotImplementedError("write your pallas kernel here")


# speedup vs baseline: 12.1876x; 12.1876x over previous
import functools

import jax
import jax.numpy as jnp
from jax import lax
from jax.experimental import pallas as pl
from jax.experimental.pallas import tpu as pltpu
from jax.experimental.pallas import tpu_sc as plsc

NC = 2
NS = 16
NW = NC * NS
CHUNK = 128
EPS = 1e-5


def _sc_mesh():
    return plsc.VectorSubcoreMesh(core_axis_name="c", subcore_axis_name="s",
                                  num_cores=NC, num_subcores=NS)


def _make_deg(e_pad, n_pad):
    steps = e_pad // (NW * CHUNK)
    stripe = n_pad // NS

    @functools.partial(
        pl.kernel,
        out_type=jax.ShapeDtypeStruct((NC * n_pad,), jnp.float32),
        mesh=_sc_mesh(),
        scratch_types=[
            pltpu.VMEM((CHUNK,), jnp.int32),
            pltpu.VMEM((CHUNK,), jnp.float32),
            pltpu.VMEM_SHARED((n_pad,), jnp.float32),
            pltpu.SemaphoreType.DMA,
        ],
    )
    def deg_kernel(dst_hbm, out_hbm, didx, ones, acc, sem):
        c = lax.axis_index("c")
        s = lax.axis_index("s")
        wid = s * NC + c

        one16 = jnp.ones((16,), jnp.float32)
        zero16 = jnp.zeros((16,), jnp.float32)

        def _fill_zero(i, carry):
            ones[pl.ds(i * 16, 16)] = zero16
            return carry

        lax.fori_loop(0, CHUNK // 16, _fill_zero, 0)

        def _zacc(i, carry):
            pltpu.sync_copy(ones, acc.at[pl.ds(s * stripe + i * CHUNK, CHUNK)])
            return carry

        lax.fori_loop(0, stripe // CHUNK, _zacc, 0)

        def _fill_one(i, carry):
            ones[pl.ds(i * 16, 16)] = one16
            return carry

        lax.fori_loop(0, CHUNK // 16, _fill_one, 0)

        plsc.subcore_barrier()

        base0 = wid * steps * CHUNK

        def _step(t, carry):
            pltpu.sync_copy(dst_hbm.at[pl.ds(base0 + t * CHUNK, CHUNK)], didx)
            pltpu.sync_copy(ones, acc.at[didx], add=True)
            return carry

        lax.fori_loop(0, steps, _step, 0)

        plsc.subcore_barrier()

        def _out(i, carry):
            off = s * stripe + i * CHUNK
            pltpu.sync_copy(acc.at[pl.ds(off, CHUNK)],
                            out_hbm.at[pl.ds(c * n_pad + off, CHUNK)])
            return carry

        lax.fori_loop(0, stripe // CHUNK, _out, 0)

    return deg_kernel


def _make_agg(e_pad, n_pad, d):
    steps = e_pad // (NW * CHUNK)
    stripe = n_pad // NS

    @functools.partial(
        pl.kernel,
        out_type=jax.ShapeDtypeStruct((NC * n_pad, d), jnp.float32),
        mesh=_sc_mesh(),
        scratch_types=[
            pltpu.VMEM((CHUNK,), jnp.int32),
            pltpu.VMEM((CHUNK,), jnp.int32),
            pltpu.VMEM((CHUNK, d), jnp.float32),
            pltpu.VMEM_SHARED((n_pad, d), jnp.float32),
            pltpu.SemaphoreType.DMA,
        ],
        compiler_params=pltpu.CompilerParams(use_tc_tiling_on_sc=False),
    )
    def agg_kernel(src_hbm, dst_hbm, g_hbm, out_hbm, sidx, didx, rows, acc, sem):
        c = lax.axis_index("c")
        s = lax.axis_index("s")
        wid = s * NC + c

        zero16 = jnp.zeros((16,), jnp.float32)
        vecs_per_row = d // 16

        def _zrow(i, carry):
            r = i // vecs_per_row
            q = i % vecs_per_row
            rows[r, pl.ds(q * 16, 16)] = zero16
            return carry

        lax.fori_loop(0, CHUNK * vecs_per_row, _zrow, 0)

        def _zacc(i, carry):
            pltpu.sync_copy(rows, acc.at[pl.ds(s * stripe + i * CHUNK, CHUNK)])
            return carry

        lax.fori_loop(0, stripe // CHUNK, _zacc, 0)

        plsc.subcore_barrier()

        base0 = wid * steps * CHUNK

        def _step(t, carry):
            base = base0 + t * CHUNK
            pltpu.sync_copy(src_hbm.at[pl.ds(base, CHUNK)], sidx)
            pltpu.sync_copy(dst_hbm.at[pl.ds(base, CHUNK)], didx)
            pltpu.async_copy(g_hbm.at[sidx], rows, sem).wait()
            pltpu.sync_copy(rows, acc.at[didx], add=True)
            return carry

        lax.fori_loop(0, steps, _step, 0)

        plsc.subcore_barrier()

        def _out(i, carry):
            off = s * stripe + i * CHUNK
            pltpu.sync_copy(acc.at[pl.ds(off, CHUNK)],
                            out_hbm.at[pl.ds(c * n_pad + off, CHUNK)])
            return carry

        lax.fori_loop(0, stripe // CHUNK, _out, 0)

    return agg_kernel


def _tc1_body(n, degp, x, w1, dinv_o, g1_o):
    deg = degp[0] + degp[1] + 1.0
    dinv = lax.rsqrt(deg)
    dinv_o[...] = dinv
    g1_o[...] = jnp.dot(x[...], w1[...], preferred_element_type=jnp.float32) * dinv


def _tc2_body(n, a1, g1, dinv, b1, bn1g, bn1b, w2, wres, bres, g2_o, res_o):
    p = g1.shape[0]
    dv = dinv[...]
    s1 = dv * (a1[0] + a1[1] + g1[...]) + b1[...]
    rid = lax.broadcasted_iota(jnp.int32, (p, 1), 0)
    valid = rid < n
    s1m = jnp.where(valid, s1, 0.0)
    mean = jnp.sum(s1m, axis=0, keepdims=True) / n
    dlt = jnp.where(valid, s1 - mean, 0.0)
    var = jnp.sum(dlt * dlt, axis=0, keepdims=True) / n
    x1 = bn1g[...] * (s1 - mean) * lax.rsqrt(var + EPS) + bn1b[...]
    x1 = jnp.where(valid, jnp.maximum(x1, 0.0), 0.0)
    g2_o[...] = jnp.dot(x1, w2[...], preferred_element_type=jnp.float32) * dv
    res_o[...] = jnp.dot(x1, wres[...], preferred_element_type=jnp.float32) + bres[...]


def _tc3_body(n, a2, g2, res, dinv, b2, bn2g, bn2b, lng, lnb, w3p, g3_o):
    p = g2.shape[0]
    dv = dinv[...]
    s2 = dv * (a2[0] + a2[1] + g2[...]) + b2[...]
    rid = lax.broadcasted_iota(jnp.int32, (p, 1), 0)
    valid = rid < n
    s2m = jnp.where(valid, s2, 0.0)
    mean = jnp.sum(s2m, axis=0, keepdims=True) / n
    dlt = jnp.where(valid, s2 - mean, 0.0)
    var = jnp.sum(dlt * dlt, axis=0, keepdims=True) / n
    x2 = bn2g[...] * (s2 - mean) * lax.rsqrt(var + EPS) + bn2b[...]
    x2 = jnp.where(valid, jnp.maximum(x2, 0.0), 0.0)
    xr = res[...] + x2
    m = jnp.mean(xr, axis=1, keepdims=True)
    v = jnp.mean((xr - m) * (xr - m), axis=1, keepdims=True)
    xr = lng[...] * (xr - m) * lax.rsqrt(v + EPS) + lnb[...]
    xr = jnp.where(valid, jnp.maximum(xr, 0.0), 0.0)
    g3_o[...] = jnp.dot(xr, w3p[...], preferred_element_type=jnp.float32) * dv


def _tc4_body(a3, g3, dinv, b3p, out_o):
    p, w = g3.shape
    o = dinv[...] * (a3[0] + a3[1] + g3[...]) + b3p[...]
    cid = lax.broadcasted_iota(jnp.int32, (p, w), 1)
    cm = cid < 2
    om = jnp.where(cm, o, -jnp.inf)
    mx = jnp.max(om, axis=1, keepdims=True)
    e = jnp.where(cm, jnp.exp(o - mx), 0.0)
    lse = mx + jnp.log(jnp.sum(e, axis=1, keepdims=True))
    out_o[...] = o - lse


def _f32(*shapes):
    return [jax.ShapeDtypeStruct(sh, jnp.float32) for sh in shapes]


def kernel(x, edge_index, W1, b1, bn1_g, bn1_b, W2, b2, bn2_g, bn2_b,
           Wres, bres, ln_g, ln_b, W3, b3):
    n, d0 = x.shape
    e = edge_index.shape[1]
    d1 = W2.shape[1]
    d3 = 16
    p = -(-n // (16 * NS)) * (16 * NS)
    e_pad = -(-e // (NW * CHUNK)) * (NW * CHUNK)

    src = jnp.concatenate(
        [edge_index[0], jnp.zeros((e_pad - e,), jnp.int32)])
    dst = jnp.concatenate(
        [edge_index[1], jnp.full((e_pad - e,), n, jnp.int32)])
    xp = jnp.pad(x, ((0, p - n), (0, 0)))
    w3p = jnp.pad(W3, ((0, 0), (0, d3 - W3.shape[1])))
    b3p = jnp.pad(b3, (0, d3 - b3.shape[0]))

    degp = _make_deg(e_pad, p)(dst).reshape(NC, p, 1)

    dinv, g1 = pl.pallas_call(
        functools.partial(_tc1_body, n),
        out_shape=_f32((p, 1), (p, d0)),
    )(degp, xp, W1)

    a1 = _make_agg(e_pad, p, d0)(src, dst, g1).reshape(NC, p, d0)

    g2, res = pl.pallas_call(
        functools.partial(_tc2_body, n),
        out_shape=_f32((p, d1), (p, d1)),
    )(a1, g1, dinv, b1, bn1_g, bn1_b, W2, Wres, bres)

    a2 = _make_agg(e_pad, p, d1)(src, dst, g2).reshape(NC, p, d1)

    g3 = pl.pallas_call(
        functools.partial(_tc3_body, n),
        out_shape=_f32((p, d3))[0],
    )(a2, g2, res, dinv, b2, bn2_g, bn2_b, ln_g, ln_b, w3p)

    a3 = _make_agg(e_pad, p, d3)(src, dst, g3).reshape(NC, p, d3)

    outp = pl.pallas_call(
        _tc4_body,
        out_shape=_f32((p, d3))[0],
    )(a3, g3, dinv, b3p)

    return outp[:n, :2]
